# baseline (speedup 1.0000x reference)
import functools

import jax
import jax.numpy as jnp
from jax.experimental import pallas as pl
from jax.experimental.pallas import tpu as pltpu

D_MODEL = 1024
D_FF = 2048
NUM_E = 8
D_FOOT = 128
DRUG_DIM = 256
N_DRUGS = 4096
N_TOK = 2048
B_TGT = 128

TOK_TILE = 256
N_TILES = N_TOK // TOK_TILE

_F32 = jnp.float32
_BF16 = jnp.bfloat16


def _bdot(a, b):
    return jax.lax.dot_general(a, b, (((1,), (0,)), ((), ())),
                               preferred_element_type=_F32)


def _smoother_body(x_ref, win_ref, bin_ref, wout_ref, bout_ref, gamma_ref,
                   beta_ref, wform_ref, wgate_ref, wfg_ref, wstats_ref,
                   jbias_ref, jac_ref, zr_out, zrb_out, form_out, gp_out):
    x = x_ref[...]
    h = jax.nn.gelu(_bdot(x.astype(_BF16), win_ref[...]) + bin_ref[...])
    z = x + _bdot(h.astype(_BF16), wout_ref[...]) + bout_ref[...]
    mu = jnp.mean(z, axis=-1, keepdims=True)
    var = jnp.mean((z - mu) ** 2, axis=-1, keepdims=True)
    zr = gamma_ref[...] * (z - mu) * jax.lax.rsqrt(var + 1e-5) + beta_ref[...]
    zrb = zr.astype(_BF16)
    form = jnp.tanh(_bdot(zrb, wform_ref[...]))
    fb = form.astype(_BF16)
    sd = jnp.sqrt(var + 1e-5)
    mu_b = mu.astype(_BF16).astype(_F32)
    sd_b = sd.astype(_BF16).astype(_F32)
    ws = wstats_ref[...].astype(_BF16).astype(_F32)
    logits = (_bdot(zrb, wgate_ref[...])
              + _bdot(fb, wfg_ref[...])
              + mu_b * ws[0:1, :] + sd_b * ws[1:2, :]
              + jac_ref[0, 0] * jbias_ref[...])
    lane = jax.lax.broadcasted_iota(jnp.int32, logits.shape, 1)
    m1 = jnp.max(logits, axis=-1, keepdims=True)
    i1 = jnp.min(jnp.where(logits == m1, lane, NUM_E), axis=-1, keepdims=True)
    l2 = jnp.where(lane == i1, -jnp.inf, logits)
    m2 = jnp.max(l2, axis=-1, keepdims=True)
    i2 = jnp.min(jnp.where(l2 == m2, lane, NUM_E), axis=-1, keepdims=True)
    e2 = jnp.exp(m2 - m1)
    denom = 1.0 + e2
    p1 = 1.0 / denom
    p2 = e2 / denom
    gp = jnp.where(lane == i1, p1, 0.0) + jnp.where(lane == i2, p2, 0.0)
    zr_out[...] = zr
    zrb_out[...] = zrb
    form_out[...] = fb
    gp_out[...] = gp


def _expert_body(gp_ref, zb_ref, w1_ref, w2_ref, out_ref):
    e = pl.program_id(1)
    h = jax.nn.gelu(_bdot(zb_ref[...], w1_ref[0]))
    y = _bdot(h.astype(_BF16), w2_ref[0])
    yb = y.astype(_BF16).astype(_F32)
    lane = jax.lax.broadcasted_iota(jnp.int32, gp_ref.shape, 1)
    pe = jnp.sum(jnp.where(lane == e, gp_ref[...], 0.0), axis=-1, keepdims=True)
    pb = pe.astype(_BF16).astype(_F32)
    acc = pb * yb

    @pl.when(e == 0)
    def _():
        out_ref[...] = acc

    @pl.when(e > 0)
    def _():
        out_ref[...] += acc


def _gather_body(idx_ref, df_ref, out_ref):
    out_ref[...] = df_ref[...]


def _scores_body(tdf_ref, wdrug_ref, wrole_ref, et_ref, form_ref, out_ref):
    tdf = tdf_ref[...]
    dp = _bdot(tdf, wdrug_ref[...])
    dpb = dp.astype(_BF16)
    rt = jnp.tanh(_bdot(tdf, wrole_ref[...]))
    rtb = rt.astype(_BF16)
    etb = et_ref[...].astype(_BF16)
    dn = (((1,), (1,)), ((), ()))
    out_ref[...] = (
        jax.lax.dot_general(etb, dpb, dn, preferred_element_type=_F32)
        + jax.lax.dot_general(form_ref[...], rtb, dn,
                              preferred_element_type=_F32))


def kernel(pillar_x, cross_floor_jaccard, drug_features, target_drug_indices,
           W_in, b_in, W_out, b_out, gamma, beta, W_form, W_role, W_stats,
           W_gate, W_fg, j_bias, W1, W2, W_drug):
    f32 = _F32
    b_in2 = b_in.reshape(1, D_MODEL)
    b_out2 = b_out.reshape(1, D_MODEL)
    gamma2 = gamma.reshape(1, D_MODEL)
    beta2 = beta.reshape(1, D_MODEL)
    jbias2 = j_bias.reshape(1, NUM_E)
    jac2 = cross_floor_jaccard.reshape(1, 1).astype(f32)

    full = lambda shape: pl.BlockSpec(shape, lambda i: (0,) * len(shape))
    zr, zrb, form, gate_probs = pl.pallas_call(
        _smoother_body,
        grid=(N_TILES,),
        in_specs=[
            pl.BlockSpec((TOK_TILE, D_MODEL), lambda i: (i, 0)),
            full((D_MODEL, D_MODEL)),
            full((1, D_MODEL)),
            full((D_MODEL, D_MODEL)),
            full((1, D_MODEL)),
            full((1, D_MODEL)),
            full((1, D_MODEL)),
            full((D_MODEL, D_FOOT)),
            full((D_MODEL, NUM_E)),
            full((D_FOOT, NUM_E)),
            full((2, NUM_E)),
            full((1, NUM_E)),
            full((1, 1)),
        ],
        out_specs=[
            pl.BlockSpec((TOK_TILE, D_MODEL), lambda i: (i, 0)),
            pl.BlockSpec((TOK_TILE, D_MODEL), lambda i: (i, 0)),
            pl.BlockSpec((TOK_TILE, D_FOOT), lambda i: (i, 0)),
            pl.BlockSpec((TOK_TILE, NUM_E), lambda i: (i, 0)),
        ],
        out_shape=[
            jax.ShapeDtypeStruct((N_TOK, D_MODEL), f32),
            jax.ShapeDtypeStruct((N_TOK, D_MODEL), _BF16),
            jax.ShapeDtypeStruct((N_TOK, D_FOOT), _BF16),
            jax.ShapeDtypeStruct((N_TOK, NUM_E), f32),
        ],
    )(pillar_x, W_in.astype(_BF16), b_in2, W_out.astype(_BF16), b_out2,
      gamma2, beta2, W_form.astype(_BF16), W_gate.astype(_BF16),
      W_fg.astype(_BF16), W_stats, jbias2, jac2)

    w1b = W1.astype(_BF16)
    w2b = W2.astype(_BF16)
    expert_tensor = pl.pallas_call(
        _expert_body,
        grid=(N_TILES, NUM_E),
        in_specs=[
            pl.BlockSpec((TOK_TILE, NUM_E), lambda t, e: (t, 0)),
            pl.BlockSpec((TOK_TILE, D_MODEL), lambda t, e: (t, 0)),
            pl.BlockSpec((1, D_MODEL, D_FF), lambda t, e: (e, 0, 0)),
            pl.BlockSpec((1, D_FF, D_MODEL), lambda t, e: (e, 0, 0)),
        ],
        out_specs=pl.BlockSpec((TOK_TILE, D_MODEL), lambda t, e: (t, 0)),
        out_shape=jax.ShapeDtypeStruct((N_TOK, D_MODEL), f32),
    )(gate_probs, zrb, w1b, w2b)

    df3 = drug_features.astype(_BF16).reshape(N_DRUGS, 1, DRUG_DIM)
    tdf = pl.pallas_call(
        _gather_body,
        grid_spec=pltpu.PrefetchScalarGridSpec(
            num_scalar_prefetch=1,
            grid=(B_TGT,),
            in_specs=[
                pl.BlockSpec((1, 1, DRUG_DIM), lambda i, idx: (idx[i], 0, 0)),
            ],
            out_specs=pl.BlockSpec((1, 1, DRUG_DIM), lambda i, idx: (i, 0, 0)),
        ),
        out_shape=jax.ShapeDtypeStruct((B_TGT, 1, DRUG_DIM), _BF16),
    )(target_drug_indices, df3).reshape(B_TGT, DRUG_DIM)

    scores = pl.pallas_call(
        _scores_body,
        grid=(N_TILES,),
        in_specs=[
            full((B_TGT, DRUG_DIM)),
            full((DRUG_DIM, D_MODEL)),
            full((DRUG_DIM, D_FOOT)),
            pl.BlockSpec((TOK_TILE, D_MODEL), lambda i: (i, 0)),
            pl.BlockSpec((TOK_TILE, D_FOOT), lambda i: (i, 0)),
        ],
        out_specs=pl.BlockSpec((TOK_TILE, B_TGT), lambda i: (i, 0)),
        out_shape=jax.ShapeDtypeStruct((N_TOK, B_TGT), f32),
    )(tdf, W_drug.astype(_BF16), W_role.astype(_BF16), expert_tensor, form)

    return (scores, gate_probs, expert_tensor)


# speedup vs baseline: 1.1475x; 1.1475x over previous
import functools

import jax
import jax.numpy as jnp
from jax import lax
from jax.experimental import pallas as pl
from jax.experimental.pallas import tpu as pltpu
from jax.experimental.pallas import tpu_sc as plsc

D_MODEL = 1024
D_FF = 2048
NUM_E = 8
D_FOOT = 128
DRUG_DIM = 256
N_DRUGS = 4096
N_TOK = 2048
B_TGT = 128

TOK_TILE = 256
N_TILES = N_TOK // TOK_TILE

P_TILE = 128
N_PTILES = N_TOK * 2 // P_TILE + NUM_E
P_ROWS = N_PTILES * P_TILE

N_WORKERS = 32
TOK_W = N_TOK // N_WORKERS
SUB_W = 32

_F32 = jnp.float32
_BF16 = jnp.bfloat16
_I32 = jnp.int32


def _bdot(a, b):
    return jax.lax.dot_general(a, b, (((1,), (0,)), ((), ())),
                               preferred_element_type=_F32)


def _smoother_body(x_ref, win_ref, bin_ref, wout_ref, bout_ref, gamma_ref,
                   beta_ref, wform_ref, wgate_ref, wfg_ref, wstats_ref,
                   jbias_ref, jac_ref, zrb_out, form_out, gp_out, i1_out,
                   i2_out, p1_out, p2_out, sel1_out, sel2_out, cnt1_out,
                   cnt2_out, carry_ref):
    t = pl.program_id(0)
    x = x_ref[...]
    h = jax.nn.gelu(_bdot(x.astype(_BF16), win_ref[...]) + bin_ref[...])
    z = x + _bdot(h.astype(_BF16), wout_ref[...]) + bout_ref[...]
    mu = jnp.mean(z, axis=-1, keepdims=True)
    var = jnp.mean((z - mu) ** 2, axis=-1, keepdims=True)
    zr = gamma_ref[...] * (z - mu) * jax.lax.rsqrt(var + 1e-5) + beta_ref[...]
    zrb = zr.astype(_BF16)
    form = jnp.tanh(_bdot(zrb, wform_ref[...]))
    fb = form.astype(_BF16)
    sd = jnp.sqrt(var + 1e-5)
    mu_b = mu.astype(_BF16).astype(_F32)
    sd_b = sd.astype(_BF16).astype(_F32)
    ws = wstats_ref[...].astype(_BF16).astype(_F32)
    logits = (_bdot(zrb, wgate_ref[...])
              + _bdot(fb, wfg_ref[...])
              + mu_b * ws[0:1, :] + sd_b * ws[1:2, :]
              + jac_ref[0, 0] * jbias_ref[...])
    lane = jax.lax.broadcasted_iota(_I32, logits.shape, 1)
    m1 = jnp.max(logits, axis=-1, keepdims=True)
    i1 = jnp.min(jnp.where(logits == m1, lane, NUM_E), axis=-1, keepdims=True)
    l2 = jnp.where(lane == i1, -jnp.inf, logits)
    m2 = jnp.max(l2, axis=-1, keepdims=True)
    i2 = jnp.min(jnp.where(l2 == m2, lane, NUM_E), axis=-1, keepdims=True)
    e2 = jnp.exp(m2 - m1)
    denom = 1.0 + e2
    p1 = 1.0 / denom
    p2 = e2 / denom
    gp = jnp.where(lane == i1, p1, 0.0) + jnp.where(lane == i2, p2, 0.0)

    @pl.when(t == 0)
    def _():
        carry_ref[...] = jnp.zeros_like(carry_ref)

    oh1 = (lane == i1).astype(_BF16)
    oh2 = (lane == i2).astype(_BF16)
    rio = jax.lax.broadcasted_iota(_I32, (TOK_TILE, TOK_TILE), 0)
    cio = jax.lax.broadcasted_iota(_I32, (TOK_TILE, TOK_TILE), 1)
    tri = (cio < rio).astype(_BF16)
    ecs1 = _bdot(tri, oh1)
    ecs2 = _bdot(tri, oh2)
    carry = carry_ref[...]
    oh1f = oh1.astype(_F32)
    oh2f = oh2.astype(_F32)
    sel1 = jnp.sum(oh1f * (ecs1 + carry[0:1, :]), axis=1, keepdims=True)
    sel2 = jnp.sum(oh2f * (ecs2 + carry[1:2, :]), axis=1, keepdims=True)
    new1 = carry[0:1, :] + jnp.sum(oh1f, axis=0, keepdims=True)
    new2 = carry[1:2, :] + jnp.sum(oh2f, axis=0, keepdims=True)
    carry_ref[...] = jnp.concatenate([new1, new2], axis=0)

    zrb_out[...] = zrb
    form_out[...] = fb
    gp_out[...] = gp
    i1_out[...] = i1
    i2_out[...] = i2
    p1_out[...] = p1.astype(_BF16).astype(_F32)
    p2_out[...] = p2.astype(_BF16).astype(_F32)
    sel1_out[...] = sel1.astype(_I32)
    sel2_out[...] = sel2.astype(_I32)
    cnt1_out[...] = new1
    cnt2_out[...] = new2


def _route_body(i1_ref, i2_ref, sel1_ref, sel2_ref, cnt1_ref, cnt2_ref,
                r1_out, r2_out, te_out):
    cnt1 = cnt1_ref[...]
    cnt = cnt1 + cnt2_ref[...]
    padded = jnp.ceil(cnt / P_TILE) * P_TILE
    cols = []
    acc = jnp.zeros((1, 1), _F32)
    for e in range(NUM_E):
        cols.append(acc)
        acc = acc + padded[0:1, e:e + 1]
    pad_off = jnp.concatenate(cols, axis=1)
    lane = jax.lax.broadcasted_iota(_I32, (N_TOK, NUM_E), 1)
    i1 = i1_ref[...]
    i2 = i2_ref[...]
    off1 = jnp.sum(jnp.where(lane == i1, pad_off, 0.0), axis=1, keepdims=True)
    off2 = jnp.sum(jnp.where(lane == i2, pad_off + cnt1, 0.0), axis=1,
                   keepdims=True)
    r1_out[...] = (off1 + sel1_ref[...].astype(_F32)).astype(_I32)
    r2_out[...] = (off2 + sel2_ref[...].astype(_F32)).astype(_I32)
    tpos = (jax.lax.broadcasted_iota(_I32, (1, N_PTILES), 1)
            .astype(_F32) * P_TILE)
    te = jnp.zeros((1, N_PTILES), _F32)
    for e in range(NUM_E):
        te = te + jnp.where(pad_off[0:1, e:e + 1] <= tpos, 1.0, 0.0)
    te_out[...] = jnp.clip(te - 1.0, 0, NUM_E - 1).astype(_I32)


def _expert_body(te_ref, z_ref, p_ref, w1_ref, w2_ref, out_ref):
    del te_ref
    h = jax.nn.gelu(_bdot(z_ref[...].astype(_BF16), w1_ref[0]))
    y = _bdot(h.astype(_BF16), w2_ref[0])
    out_ref[...] = p_ref[...] * y.astype(_BF16).astype(_F32)


def _gather_body(idx_ref, df_ref, out_ref):
    out_ref[...] = df_ref[...]


def _scores_body(tdf_ref, wdrug_ref, wrole_ref, et_ref, form_ref, out_ref):
    tdf = tdf_ref[...]
    dp = _bdot(tdf, wdrug_ref[...])
    dpb = dp.astype(_BF16)
    rt = jnp.tanh(_bdot(tdf, wrole_ref[...]))
    rtb = rt.astype(_BF16)
    etb = et_ref[...].astype(_BF16)
    dn = (((1,), (1,)), ((), ()))
    out_ref[...] = (
        jax.lax.dot_general(etb, dpb, dn, preferred_element_type=_F32)
        + jax.lax.dot_general(form_ref[...], rtb, dn,
                              preferred_element_type=_F32))


def _dispatch_body(z_hbm, r1_hbm, r2_hbm, p1_hbm, p2_hbm, zs_hbm, ps_hbm,
                   idx1_v, idx2_v, p1_v, p2_v, rows_v, sem1, sem2, sem3,
                   sem4):
    wid = lax.axis_index("s") * 2 + lax.axis_index("c")
    base = wid * TOK_W
    pltpu.sync_copy(r1_hbm.at[pl.ds(base, TOK_W)], idx1_v)
    pltpu.sync_copy(r2_hbm.at[pl.ds(base, TOK_W)], idx2_v)
    pltpu.sync_copy(p1_hbm.at[pl.ds(base, TOK_W)], p1_v)
    pltpu.sync_copy(p2_hbm.at[pl.ds(base, TOK_W)], p2_v)
    pltpu.sync_copy(z_hbm.at[pl.ds(base, TOK_W)], rows_v)
    c1 = pltpu.async_copy(rows_v, zs_hbm.at[idx1_v], sem1)
    c2 = pltpu.async_copy(rows_v, zs_hbm.at[idx2_v], sem2)
    c3 = pltpu.async_copy(p1_v, ps_hbm.at[idx1_v], sem3)
    c4 = pltpu.async_copy(p2_v, ps_hbm.at[idx2_v], sem4)
    c1.wait()
    c2.wait()
    c3.wait()
    c4.wait()


@functools.lru_cache(maxsize=None)
def _dispatch_sc():
    mesh = plsc.VectorSubcoreMesh(core_axis_name="c", subcore_axis_name="s")
    return pl.kernel(
        _dispatch_body,
        out_type=[
            jax.ShapeDtypeStruct((P_ROWS, D_MODEL), jnp.float32),
            jax.ShapeDtypeStruct((P_ROWS,), jnp.float32),
        ],
        mesh=mesh,
        scratch_types=[
            pltpu.VMEM((TOK_W,), _I32),
            pltpu.VMEM((TOK_W,), _I32),
            pltpu.VMEM((TOK_W,), _F32),
            pltpu.VMEM((TOK_W,), _F32),
            pltpu.VMEM((TOK_W, D_MODEL), _F32),
            pltpu.SemaphoreType.DMA,
            pltpu.SemaphoreType.DMA,
            pltpu.SemaphoreType.DMA,
            pltpu.SemaphoreType.DMA,
        ],
    )


def _combine_body(y_hbm, r1_hbm, r2_hbm, et_hbm, i1_v, i2_v, a_v, b_v, o_v,
                  sema, semb):
    wid = lax.axis_index("s") * 2 + lax.axis_index("c")
    for sub in range(TOK_W // SUB_W):
        base = wid * TOK_W + sub * SUB_W
        pltpu.sync_copy(r1_hbm.at[pl.ds(base, SUB_W)], i1_v)
        pltpu.sync_copy(r2_hbm.at[pl.ds(base, SUB_W)], i2_v)
        ca = pltpu.async_copy(y_hbm.at[i1_v], a_v, sema)
        cb = pltpu.async_copy(y_hbm.at[i2_v], b_v, semb)
        ca.wait()
        cb.wait()
        for j in range(SUB_W):
            arow = a_v.at[j]
            brow = b_v.at[j]
            orow = o_v.at[j]

            def chunk(k, _, arow=arow, brow=brow, orow=orow):
                sl = pl.ds(k * 16, 16)
                orow[sl] = arow[sl] + brow[sl]
                return 0

            lax.fori_loop(0, D_MODEL // 16, chunk, 0)
        pltpu.sync_copy(o_v, et_hbm.at[pl.ds(base, SUB_W)])


@functools.lru_cache(maxsize=None)
def _combine_sc():
    mesh = plsc.VectorSubcoreMesh(core_axis_name="c", subcore_axis_name="s")
    return pl.kernel(
        _combine_body,
        out_type=jax.ShapeDtypeStruct((N_TOK, D_MODEL), jnp.float32),
        mesh=mesh,
        scratch_types=[
            pltpu.VMEM((SUB_W,), _I32),
            pltpu.VMEM((SUB_W,), _I32),
            pltpu.VMEM((SUB_W, D_MODEL), _F32),
            pltpu.VMEM((SUB_W, D_MODEL), _F32),
            pltpu.VMEM((SUB_W, D_MODEL), _F32),
            pltpu.SemaphoreType.DMA,
            pltpu.SemaphoreType.DMA,
        ],
    )


def kernel(pillar_x, cross_floor_jaccard, drug_features, target_drug_indices,
           W_in, b_in, W_out, b_out, gamma, beta, W_form, W_role, W_stats,
           W_gate, W_fg, j_bias, W1, W2, W_drug):
    f32 = _F32
    b_in2 = b_in.reshape(1, D_MODEL)
    b_out2 = b_out.reshape(1, D_MODEL)
    gamma2 = gamma.reshape(1, D_MODEL)
    beta2 = beta.reshape(1, D_MODEL)
    jbias2 = j_bias.reshape(1, NUM_E)
    jac2 = cross_floor_jaccard.reshape(1, 1).astype(f32)

    full = lambda shape: pl.BlockSpec(shape, lambda i: (0,) * len(shape))
    tile = lambda w: pl.BlockSpec((TOK_TILE, w), lambda i: (i, 0))
    (zrb, form, gate_probs, i1, i2, p1b, p2b, sel1, sel2, cnt1, cnt2) = (
        pl.pallas_call(
            _smoother_body,
            grid=(N_TILES,),
            in_specs=[
                tile(D_MODEL),
                full((D_MODEL, D_MODEL)),
                full((1, D_MODEL)),
                full((D_MODEL, D_MODEL)),
                full((1, D_MODEL)),
                full((1, D_MODEL)),
                full((1, D_MODEL)),
                full((D_MODEL, D_FOOT)),
                full((D_MODEL, NUM_E)),
                full((D_FOOT, NUM_E)),
                full((2, NUM_E)),
                full((1, NUM_E)),
                full((1, 1)),
            ],
            out_specs=[
                tile(D_MODEL),
                tile(D_FOOT),
                tile(NUM_E),
                tile(1),
                tile(1),
                tile(1),
                tile(1),
                tile(1),
                tile(1),
                pl.BlockSpec((1, NUM_E), lambda i: (0, 0)),
                pl.BlockSpec((1, NUM_E), lambda i: (0, 0)),
            ],
            out_shape=[
                jax.ShapeDtypeStruct((N_TOK, D_MODEL), _BF16),
                jax.ShapeDtypeStruct((N_TOK, D_FOOT), _BF16),
                jax.ShapeDtypeStruct((N_TOK, NUM_E), f32),
                jax.ShapeDtypeStruct((N_TOK, 1), _I32),
                jax.ShapeDtypeStruct((N_TOK, 1), _I32),
                jax.ShapeDtypeStruct((N_TOK, 1), f32),
                jax.ShapeDtypeStruct((N_TOK, 1), f32),
                jax.ShapeDtypeStruct((N_TOK, 1), _I32),
                jax.ShapeDtypeStruct((N_TOK, 1), _I32),
                jax.ShapeDtypeStruct((1, NUM_E), f32),
                jax.ShapeDtypeStruct((1, NUM_E), f32),
            ],
            scratch_shapes=[pltpu.VMEM((2, NUM_E), f32)],
        )(pillar_x, W_in.astype(_BF16), b_in2, W_out.astype(_BF16), b_out2,
          gamma2, beta2, W_form.astype(_BF16), W_gate.astype(_BF16),
          W_fg.astype(_BF16), W_stats, jbias2, jac2))
    zr32 = zrb.astype(f32)

    r1, r2, te = pl.pallas_call(
        _route_body,
        grid=(1,),
        in_specs=[
            pl.BlockSpec((N_TOK, 1), lambda i: (0, 0)),
            pl.BlockSpec((N_TOK, 1), lambda i: (0, 0)),
            pl.BlockSpec((N_TOK, 1), lambda i: (0, 0)),
            pl.BlockSpec((N_TOK, 1), lambda i: (0, 0)),
            pl.BlockSpec((1, NUM_E), lambda i: (0, 0)),
            pl.BlockSpec((1, NUM_E), lambda i: (0, 0)),
        ],
        out_specs=[
            pl.BlockSpec((N_TOK, 1), lambda i: (0, 0)),
            pl.BlockSpec((N_TOK, 1), lambda i: (0, 0)),
            pl.BlockSpec((1, N_PTILES), lambda i: (0, 0)),
        ],
        out_shape=[
            jax.ShapeDtypeStruct((N_TOK, 1), _I32),
            jax.ShapeDtypeStruct((N_TOK, 1), _I32),
            jax.ShapeDtypeStruct((1, N_PTILES), _I32),
        ],
    )(i1, i2, sel1, sel2, cnt1, cnt2)

    r1f = r1.reshape(N_TOK)
    r2f = r2.reshape(N_TOK)
    z_sorted, p_sorted = _dispatch_sc()(zr32, r1f, r2f, p1b.reshape(N_TOK),
                                        p2b.reshape(N_TOK))

    w1b = W1.astype(_BF16)
    w2b = W2.astype(_BF16)
    y_sorted = pl.pallas_call(
        _expert_body,
        grid_spec=pltpu.PrefetchScalarGridSpec(
            num_scalar_prefetch=1,
            grid=(N_PTILES,),
            in_specs=[
                pl.BlockSpec((P_TILE, D_MODEL), lambda i, te: (i, 0)),
                pl.BlockSpec((P_TILE, 1), lambda i, te: (i, 0)),
                pl.BlockSpec((1, D_MODEL, D_FF), lambda i, te: (te[i], 0, 0)),
                pl.BlockSpec((1, D_FF, D_MODEL), lambda i, te: (te[i], 0, 0)),
            ],
            out_specs=pl.BlockSpec((P_TILE, D_MODEL), lambda i, te: (i, 0)),
        ),
        out_shape=jax.ShapeDtypeStruct((P_ROWS, D_MODEL), f32),
    )(te.reshape(N_PTILES), z_sorted, p_sorted.reshape(P_ROWS, 1), w1b, w2b)

    expert_tensor = _combine_sc()(y_sorted, r1f, r2f)

    df3 = drug_features.astype(_BF16).reshape(N_DRUGS, 1, DRUG_DIM)
    tdf = pl.pallas_call(
        _gather_body,
        grid_spec=pltpu.PrefetchScalarGridSpec(
            num_scalar_prefetch=1,
            grid=(B_TGT,),
            in_specs=[
                pl.BlockSpec((1, 1, DRUG_DIM), lambda i, idx: (idx[i], 0, 0)),
            ],
            out_specs=pl.BlockSpec((1, 1, DRUG_DIM), lambda i, idx: (i, 0, 0)),
        ),
        out_shape=jax.ShapeDtypeStruct((B_TGT, 1, DRUG_DIM), _BF16),
    )(target_drug_indices, df3).reshape(B_TGT, DRUG_DIM)

    scores = pl.pallas_call(
        _scores_body,
        grid=(N_TILES,),
        in_specs=[
            full((B_TGT, DRUG_DIM)),
            full((DRUG_DIM, D_MODEL)),
            full((DRUG_DIM, D_FOOT)),
            tile(D_MODEL),
            tile(D_FOOT),
        ],
        out_specs=pl.BlockSpec((TOK_TILE, B_TGT), lambda i: (i, 0)),
        out_shape=jax.ShapeDtypeStruct((N_TOK, B_TGT), f32),
    )(tdf, W_drug.astype(_BF16), W_role.astype(_BF16), expert_tensor, form)

    return (scores, gate_probs, expert_tensor)


# speedup vs baseline: 1.2085x; 1.0531x over previous
import functools

import jax
import jax.numpy as jnp
from jax import lax
from jax.experimental import pallas as pl
from jax.experimental.pallas import tpu as pltpu
from jax.experimental.pallas import tpu_sc as plsc

D_MODEL = 1024
D_FF = 2048
NUM_E = 8
D_FOOT = 128
DRUG_DIM = 256
N_DRUGS = 4096
N_TOK = 2048
B_TGT = 128

TOK_TILE = 256
N_TILES = N_TOK // TOK_TILE

P_TILE = 128
N_PTILES = N_TOK * 2 // P_TILE + NUM_E
P_ROWS = N_PTILES * P_TILE

N_WORKERS = 32
TOK_W = N_TOK // N_WORKERS
SUB_W = 32

_F32 = jnp.float32
_BF16 = jnp.bfloat16
_I32 = jnp.int32


def _bdot(a, b):
    return jax.lax.dot_general(a, b, (((1,), (0,)), ((), ())),
                               preferred_element_type=_F32)


def _smoother_body(x_ref, win_ref, bin_ref, wout_ref, bout_ref, gamma_ref,
                   beta_ref, wform_ref, wgate_ref, wfg_ref, wstats_ref,
                   jbias_ref, jac_ref, zrb_out, form_out, gp_out, i1_out,
                   i2_out, p1_out, p2_out, sel1_out, sel2_out, cnt1_out,
                   cnt2_out, carry_ref):
    t = pl.program_id(0)
    x = x_ref[...]
    h = jax.nn.gelu(_bdot(x.astype(_BF16), win_ref[...]) + bin_ref[...])
    z = x + _bdot(h.astype(_BF16), wout_ref[...]) + bout_ref[...]
    mu = jnp.mean(z, axis=-1, keepdims=True)
    var = jnp.mean((z - mu) ** 2, axis=-1, keepdims=True)
    zr = gamma_ref[...] * (z - mu) * jax.lax.rsqrt(var + 1e-5) + beta_ref[...]
    zrb = zr.astype(_BF16)
    form = jnp.tanh(_bdot(zrb, wform_ref[...]))
    fb = form.astype(_BF16)
    sd = jnp.sqrt(var + 1e-5)
    mu_b = mu.astype(_BF16).astype(_F32)
    sd_b = sd.astype(_BF16).astype(_F32)
    ws = wstats_ref[...].astype(_BF16).astype(_F32)
    logits = (_bdot(zrb, wgate_ref[...])
              + _bdot(fb, wfg_ref[...])
              + mu_b * ws[0:1, :] + sd_b * ws[1:2, :]
              + jac_ref[0, 0] * jbias_ref[...])
    lane = jax.lax.broadcasted_iota(_I32, logits.shape, 1)
    m1 = jnp.max(logits, axis=-1, keepdims=True)
    i1 = jnp.min(jnp.where(logits == m1, lane, NUM_E), axis=-1, keepdims=True)
    l2 = jnp.where(lane == i1, -jnp.inf, logits)
    m2 = jnp.max(l2, axis=-1, keepdims=True)
    i2 = jnp.min(jnp.where(l2 == m2, lane, NUM_E), axis=-1, keepdims=True)
    e2 = jnp.exp(m2 - m1)
    denom = 1.0 + e2
    p1 = 1.0 / denom
    p2 = e2 / denom
    gp = jnp.where(lane == i1, p1, 0.0) + jnp.where(lane == i2, p2, 0.0)

    @pl.when(t == 0)
    def _():
        carry_ref[...] = jnp.zeros_like(carry_ref)

    oh1 = (lane == i1).astype(_BF16)
    oh2 = (lane == i2).astype(_BF16)
    rio = jax.lax.broadcasted_iota(_I32, (TOK_TILE, TOK_TILE), 0)
    cio = jax.lax.broadcasted_iota(_I32, (TOK_TILE, TOK_TILE), 1)
    tri = (cio < rio).astype(_BF16)
    ecs1 = _bdot(tri, oh1)
    ecs2 = _bdot(tri, oh2)
    carry = carry_ref[...]
    oh1f = oh1.astype(_F32)
    oh2f = oh2.astype(_F32)
    sel1 = jnp.sum(oh1f * (ecs1 + carry[0:1, :]), axis=1, keepdims=True)
    sel2 = jnp.sum(oh2f * (ecs2 + carry[1:2, :]), axis=1, keepdims=True)
    new1 = carry[0:1, :] + jnp.sum(oh1f, axis=0, keepdims=True)
    new2 = carry[1:2, :] + jnp.sum(oh2f, axis=0, keepdims=True)
    carry_ref[...] = jnp.concatenate([new1, new2], axis=0)

    zrb_out[...] = zrb
    form_out[...] = fb
    gp_out[...] = gp
    i1_out[...] = i1
    i2_out[...] = i2
    p1_out[...] = p1.astype(_BF16).astype(_F32)
    p2_out[...] = p2.astype(_BF16).astype(_F32)
    sel1_out[...] = sel1.astype(_I32)
    sel2_out[...] = sel2.astype(_I32)
    cnt1_out[...] = new1
    cnt2_out[...] = new2


def _route_body(i1_ref, i2_ref, sel1_ref, sel2_ref, cnt1_ref, cnt2_ref,
                r1_out, r2_out, te_out):
    cnt1 = cnt1_ref[...]
    cnt = cnt1 + cnt2_ref[...]
    padded = jnp.ceil(cnt / P_TILE) * P_TILE
    cols = []
    acc = jnp.zeros((1, 1), _F32)
    for e in range(NUM_E):
        cols.append(acc)
        acc = acc + padded[0:1, e:e + 1]
    pad_off = jnp.concatenate(cols, axis=1)
    lane = jax.lax.broadcasted_iota(_I32, (N_TOK, NUM_E), 1)
    i1 = i1_ref[...]
    i2 = i2_ref[...]
    off1 = jnp.sum(jnp.where(lane == i1, pad_off, 0.0), axis=1, keepdims=True)
    off2 = jnp.sum(jnp.where(lane == i2, pad_off + cnt1, 0.0), axis=1,
                   keepdims=True)
    r1_out[...] = (off1 + sel1_ref[...].astype(_F32)).astype(_I32)
    r2_out[...] = (off2 + sel2_ref[...].astype(_F32)).astype(_I32)
    tpos = (jax.lax.broadcasted_iota(_I32, (1, N_PTILES), 1)
            .astype(_F32) * P_TILE)
    te = jnp.zeros((1, N_PTILES), _F32)
    for e in range(NUM_E):
        te = te + jnp.where(pad_off[0:1, e:e + 1] <= tpos, 1.0, 0.0)
    te_out[...] = jnp.clip(te - 1.0, 0, NUM_E - 1).astype(_I32)


def _expert_body(te_ref, z_ref, r1_ref, r2_ref, p1_ref, p2_ref, w1_ref,
                 w2_ref, out_ref):
    del te_ref
    h = jax.nn.gelu(_bdot(z_ref[...].astype(_BF16), w1_ref[0]))
    y = _bdot(h.astype(_BF16), w2_ref[0])
    slot = (jax.lax.broadcasted_iota(_I32, (P_TILE, N_TOK), 0)
            + pl.program_id(0) * P_TILE)
    oh1 = (r1_ref[...] == slot).astype(_BF16)
    oh2 = (r2_ref[...] == slot).astype(_BF16)
    p_tile = (_bdot(oh1, p1_ref[...].astype(_BF16))
              + _bdot(oh2, p2_ref[...].astype(_BF16)))
    out_ref[...] = p_tile * y.astype(_BF16).astype(_F32)


def _gather_body(idx_ref, df_ref, out_ref):
    out_ref[...] = df_ref[...]


def _scores_body(tdf_ref, wdrug_ref, wrole_ref, et_ref, form_ref, out_ref):
    tdf = tdf_ref[...]
    dp = _bdot(tdf, wdrug_ref[...])
    dpb = dp.astype(_BF16)
    rt = jnp.tanh(_bdot(tdf, wrole_ref[...]))
    rtb = rt.astype(_BF16)
    etb = et_ref[...].astype(_BF16)
    dn = (((1,), (1,)), ((), ()))
    out_ref[...] = (
        jax.lax.dot_general(etb, dpb, dn, preferred_element_type=_F32)
        + jax.lax.dot_general(form_ref[...], rtb, dn,
                              preferred_element_type=_F32))


def _dispatch_body(z_hbm, r1_hbm, r2_hbm, zs_hbm, idx1_v, idx2_v, rows_v,
                   sem1, sem2):
    wid = lax.axis_index("s") * 2 + lax.axis_index("c")
    base = wid * TOK_W
    pltpu.sync_copy(r1_hbm.at[pl.ds(base, TOK_W)], idx1_v)
    pltpu.sync_copy(r2_hbm.at[pl.ds(base, TOK_W)], idx2_v)
    pltpu.sync_copy(z_hbm.at[pl.ds(base, TOK_W)], rows_v)
    c1 = pltpu.async_copy(rows_v, zs_hbm.at[idx1_v], sem1)
    c2 = pltpu.async_copy(rows_v, zs_hbm.at[idx2_v], sem2)
    c1.wait()
    c2.wait()


@functools.lru_cache(maxsize=None)
def _dispatch_sc():
    mesh = plsc.VectorSubcoreMesh(core_axis_name="c", subcore_axis_name="s")
    return pl.kernel(
        _dispatch_body,
        out_type=jax.ShapeDtypeStruct((P_ROWS, D_MODEL), jnp.float32),
        mesh=mesh,
        scratch_types=[
            pltpu.VMEM((TOK_W,), _I32),
            pltpu.VMEM((TOK_W,), _I32),
            pltpu.VMEM((TOK_W, D_MODEL), _F32),
            pltpu.SemaphoreType.DMA,
            pltpu.SemaphoreType.DMA,
        ],
    )


def _combine_body(y_hbm, r1_hbm, r2_hbm, et_hbm, i1_v, i2_v, a_v, b_v, o_v,
                  sema, semb):
    wid = lax.axis_index("s") * 2 + lax.axis_index("c")
    for sub in range(TOK_W // SUB_W):
        base = wid * TOK_W + sub * SUB_W
        pltpu.sync_copy(r1_hbm.at[pl.ds(base, SUB_W)], i1_v)
        pltpu.sync_copy(r2_hbm.at[pl.ds(base, SUB_W)], i2_v)
        ca = pltpu.async_copy(y_hbm.at[i1_v], a_v, sema)
        cb = pltpu.async_copy(y_hbm.at[i2_v], b_v, semb)
        ca.wait()
        cb.wait()
        for j in range(SUB_W):
            arow = a_v.at[j]
            brow = b_v.at[j]
            orow = o_v.at[j]

            def chunk(k, _, arow=arow, brow=brow, orow=orow):
                sl = pl.ds(k * 16, 16)
                orow[sl] = arow[sl] + brow[sl]
                return 0

            lax.fori_loop(0, D_MODEL // 16, chunk, 0)
        pltpu.sync_copy(o_v, et_hbm.at[pl.ds(base, SUB_W)])


@functools.lru_cache(maxsize=None)
def _combine_sc():
    mesh = plsc.VectorSubcoreMesh(core_axis_name="c", subcore_axis_name="s")
    return pl.kernel(
        _combine_body,
        out_type=jax.ShapeDtypeStruct((N_TOK, D_MODEL), jnp.float32),
        mesh=mesh,
        scratch_types=[
            pltpu.VMEM((SUB_W,), _I32),
            pltpu.VMEM((SUB_W,), _I32),
            pltpu.VMEM((SUB_W, D_MODEL), _F32),
            pltpu.VMEM((SUB_W, D_MODEL), _F32),
            pltpu.VMEM((SUB_W, D_MODEL), _F32),
            pltpu.SemaphoreType.DMA,
            pltpu.SemaphoreType.DMA,
        ],
    )


def kernel(pillar_x, cross_floor_jaccard, drug_features, target_drug_indices,
           W_in, b_in, W_out, b_out, gamma, beta, W_form, W_role, W_stats,
           W_gate, W_fg, j_bias, W1, W2, W_drug):
    f32 = _F32
    b_in2 = b_in.reshape(1, D_MODEL)
    b_out2 = b_out.reshape(1, D_MODEL)
    gamma2 = gamma.reshape(1, D_MODEL)
    beta2 = beta.reshape(1, D_MODEL)
    jbias2 = j_bias.reshape(1, NUM_E)
    jac2 = cross_floor_jaccard.reshape(1, 1).astype(f32)

    full = lambda shape: pl.BlockSpec(shape, lambda i: (0,) * len(shape))
    tile = lambda w: pl.BlockSpec((TOK_TILE, w), lambda i: (i, 0))
    (zrb, form, gate_probs, i1, i2, p1b, p2b, sel1, sel2, cnt1, cnt2) = (
        pl.pallas_call(
            _smoother_body,
            grid=(N_TILES,),
            in_specs=[
                tile(D_MODEL),
                full((D_MODEL, D_MODEL)),
                full((1, D_MODEL)),
                full((D_MODEL, D_MODEL)),
                full((1, D_MODEL)),
                full((1, D_MODEL)),
                full((1, D_MODEL)),
                full((D_MODEL, D_FOOT)),
                full((D_MODEL, NUM_E)),
                full((D_FOOT, NUM_E)),
                full((2, NUM_E)),
                full((1, NUM_E)),
                full((1, 1)),
            ],
            out_specs=[
                tile(D_MODEL),
                tile(D_FOOT),
                tile(NUM_E),
                tile(1),
                tile(1),
                tile(1),
                tile(1),
                tile(1),
                tile(1),
                pl.BlockSpec((1, NUM_E), lambda i: (0, 0)),
                pl.BlockSpec((1, NUM_E), lambda i: (0, 0)),
            ],
            out_shape=[
                jax.ShapeDtypeStruct((N_TOK, D_MODEL), _BF16),
                jax.ShapeDtypeStruct((N_TOK, D_FOOT), _BF16),
                jax.ShapeDtypeStruct((N_TOK, NUM_E), f32),
                jax.ShapeDtypeStruct((N_TOK, 1), _I32),
                jax.ShapeDtypeStruct((N_TOK, 1), _I32),
                jax.ShapeDtypeStruct((N_TOK, 1), f32),
                jax.ShapeDtypeStruct((N_TOK, 1), f32),
                jax.ShapeDtypeStruct((N_TOK, 1), _I32),
                jax.ShapeDtypeStruct((N_TOK, 1), _I32),
                jax.ShapeDtypeStruct((1, NUM_E), f32),
                jax.ShapeDtypeStruct((1, NUM_E), f32),
            ],
            scratch_shapes=[pltpu.VMEM((2, NUM_E), f32)],
        )(pillar_x, W_in.astype(_BF16), b_in2, W_out.astype(_BF16), b_out2,
          gamma2, beta2, W_form.astype(_BF16), W_gate.astype(_BF16),
          W_fg.astype(_BF16), W_stats, jbias2, jac2))
    zr32 = zrb.astype(f32)

    r1, r2, te = pl.pallas_call(
        _route_body,
        grid=(1,),
        in_specs=[
            pl.BlockSpec((N_TOK, 1), lambda i: (0, 0)),
            pl.BlockSpec((N_TOK, 1), lambda i: (0, 0)),
            pl.BlockSpec((N_TOK, 1), lambda i: (0, 0)),
            pl.BlockSpec((N_TOK, 1), lambda i: (0, 0)),
            pl.BlockSpec((1, NUM_E), lambda i: (0, 0)),
            pl.BlockSpec((1, NUM_E), lambda i: (0, 0)),
        ],
        out_specs=[
            pl.BlockSpec((N_TOK, 1), lambda i: (0, 0)),
            pl.BlockSpec((N_TOK, 1), lambda i: (0, 0)),
            pl.BlockSpec((1, N_PTILES), lambda i: (0, 0)),
        ],
        out_shape=[
            jax.ShapeDtypeStruct((N_TOK, 1), _I32),
            jax.ShapeDtypeStruct((N_TOK, 1), _I32),
            jax.ShapeDtypeStruct((1, N_PTILES), _I32),
        ],
    )(i1, i2, sel1, sel2, cnt1, cnt2)

    r1f = r1.reshape(N_TOK)
    r2f = r2.reshape(N_TOK)
    z_sorted = _dispatch_sc()(zr32, r1f, r2f)

    w1b = W1.astype(_BF16)
    w2b = W2.astype(_BF16)
    y_sorted = pl.pallas_call(
        _expert_body,
        grid_spec=pltpu.PrefetchScalarGridSpec(
            num_scalar_prefetch=1,
            grid=(N_PTILES,),
            in_specs=[
                pl.BlockSpec((P_TILE, D_MODEL), lambda i, te: (i, 0)),
                pl.BlockSpec((1, N_TOK), lambda i, te: (0, 0)),
                pl.BlockSpec((1, N_TOK), lambda i, te: (0, 0)),
                pl.BlockSpec((N_TOK, 1), lambda i, te: (0, 0)),
                pl.BlockSpec((N_TOK, 1), lambda i, te: (0, 0)),
                pl.BlockSpec((1, D_MODEL, D_FF), lambda i, te: (te[i], 0, 0)),
                pl.BlockSpec((1, D_FF, D_MODEL), lambda i, te: (te[i], 0, 0)),
            ],
            out_specs=pl.BlockSpec((P_TILE, D_MODEL), lambda i, te: (i, 0)),
        ),
        out_shape=jax.ShapeDtypeStruct((P_ROWS, D_MODEL), f32),
    )(te.reshape(N_PTILES), z_sorted, r1.reshape(1, N_TOK),
      r2.reshape(1, N_TOK), p1b, p2b, w1b, w2b)

    expert_tensor = _combine_sc()(y_sorted, r1f, r2f)

    df3 = drug_features.astype(_BF16).reshape(N_DRUGS, 1, DRUG_DIM)
    tdf = pl.pallas_call(
        _gather_body,
        grid_spec=pltpu.PrefetchScalarGridSpec(
            num_scalar_prefetch=1,
            grid=(B_TGT,),
            in_specs=[
                pl.BlockSpec((1, 1, DRUG_DIM), lambda i, idx: (idx[i], 0, 0)),
            ],
            out_specs=pl.BlockSpec((1, 1, DRUG_DIM), lambda i, idx: (i, 0, 0)),
        ),
        out_shape=jax.ShapeDtypeStruct((B_TGT, 1, DRUG_DIM), _BF16),
    )(target_drug_indices, df3).reshape(B_TGT, DRUG_DIM)

    scores = pl.pallas_call(
        _scores_body,
        grid=(N_TILES,),
        in_specs=[
            full((B_TGT, DRUG_DIM)),
            full((DRUG_DIM, D_MODEL)),
            full((DRUG_DIM, D_FOOT)),
            tile(D_MODEL),
            tile(D_FOOT),
        ],
        out_specs=pl.BlockSpec((TOK_TILE, B_TGT), lambda i: (i, 0)),
        out_shape=jax.ShapeDtypeStruct((N_TOK, B_TGT), f32),
    )(tdf, W_drug.astype(_BF16), W_role.astype(_BF16), expert_tensor, form)

    return (scores, gate_probs, expert_tensor)


# speedup vs baseline: 1.4343x; 1.1868x over previous
import functools

import jax
import jax.numpy as jnp
from jax import lax
from jax.experimental import pallas as pl
from jax.experimental.pallas import tpu as pltpu
from jax.experimental.pallas import tpu_sc as plsc

D_MODEL = 1024
D_FF = 2048
NUM_E = 8
D_FOOT = 128
DRUG_DIM = 256
N_DRUGS = 4096
N_TOK = 2048
B_TGT = 128

TOK_TILE = 256
N_TILES = N_TOK // TOK_TILE

P_TILE = 128
N_PTILES = N_TOK * 2 // P_TILE + NUM_E
P_ROWS = N_PTILES * P_TILE

N_WORKERS = 32
TOK_W = N_TOK // N_WORKERS
SUB_W = 32

_F32 = jnp.float32
_BF16 = jnp.bfloat16
_I32 = jnp.int32


def _bdot(a, b):
    return jax.lax.dot_general(a, b, (((1,), (0,)), ((), ())),
                               preferred_element_type=_F32)


def _smoother_body(x_ref, win_ref, bin_ref, wout_ref, bout_ref, gamma_ref,
                   beta_ref, wform_ref, wgate_ref, wfg_ref, wstats_ref,
                   jbias_ref, jac_ref, zrb_out, form_out, gp_out, i1_out,
                   i2_out, p1_out, p2_out, sel1_out, sel2_out, cnt1_out,
                   cnt2_out, carry_ref):
    t = pl.program_id(0)
    x = x_ref[...]
    h = jax.nn.gelu(_bdot(x.astype(_BF16), win_ref[...]) + bin_ref[...])
    z = x + _bdot(h.astype(_BF16), wout_ref[...]) + bout_ref[...]
    mu = jnp.mean(z, axis=-1, keepdims=True)
    var = jnp.mean((z - mu) ** 2, axis=-1, keepdims=True)
    zr = gamma_ref[...] * (z - mu) * jax.lax.rsqrt(var + 1e-5) + beta_ref[...]
    zrb = zr.astype(_BF16)
    form = jnp.tanh(_bdot(zrb, wform_ref[...]))
    fb = form.astype(_BF16)
    sd = jnp.sqrt(var + 1e-5)
    mu_b = mu.astype(_BF16).astype(_F32)
    sd_b = sd.astype(_BF16).astype(_F32)
    ws = wstats_ref[...].astype(_BF16).astype(_F32)
    logits = (_bdot(zrb, wgate_ref[...])
              + _bdot(fb, wfg_ref[...])
              + mu_b * ws[0:1, :] + sd_b * ws[1:2, :]
              + jac_ref[0, 0] * jbias_ref[...])
    lane = jax.lax.broadcasted_iota(_I32, logits.shape, 1)
    m1 = jnp.max(logits, axis=-1, keepdims=True)
    i1 = jnp.min(jnp.where(logits == m1, lane, NUM_E), axis=-1, keepdims=True)
    l2 = jnp.where(lane == i1, -jnp.inf, logits)
    m2 = jnp.max(l2, axis=-1, keepdims=True)
    i2 = jnp.min(jnp.where(l2 == m2, lane, NUM_E), axis=-1, keepdims=True)
    e2 = jnp.exp(m2 - m1)
    denom = 1.0 + e2
    p1 = 1.0 / denom
    p2 = e2 / denom
    gp = jnp.where(lane == i1, p1, 0.0) + jnp.where(lane == i2, p2, 0.0)

    @pl.when(t == 0)
    def _():
        carry_ref[...] = jnp.zeros_like(carry_ref)

    oh1 = (lane == i1).astype(_BF16)
    oh2 = (lane == i2).astype(_BF16)
    rio = jax.lax.broadcasted_iota(_I32, (TOK_TILE, TOK_TILE), 0)
    cio = jax.lax.broadcasted_iota(_I32, (TOK_TILE, TOK_TILE), 1)
    tri = (cio < rio).astype(_BF16)
    ecs1 = _bdot(tri, oh1)
    ecs2 = _bdot(tri, oh2)
    carry = carry_ref[...]
    oh1f = oh1.astype(_F32)
    oh2f = oh2.astype(_F32)
    sel1 = jnp.sum(oh1f * (ecs1 + carry[0:1, :]), axis=1, keepdims=True)
    sel2 = jnp.sum(oh2f * (ecs2 + carry[1:2, :]), axis=1, keepdims=True)
    new1 = carry[0:1, :] + jnp.sum(oh1f, axis=0, keepdims=True)
    new2 = carry[1:2, :] + jnp.sum(oh2f, axis=0, keepdims=True)
    carry_ref[...] = jnp.concatenate([new1, new2], axis=0)

    zrb_out[...] = zrb
    form_out[...] = fb
    gp_out[...] = gp
    i1_out[...] = i1
    i2_out[...] = i2
    p1_out[...] = p1.astype(_BF16).astype(_F32)
    p2_out[...] = p2.astype(_BF16).astype(_F32)
    sel1_out[...] = sel1.astype(_I32)
    sel2_out[...] = sel2.astype(_I32)
    cnt1_out[...] = new1
    cnt2_out[...] = new2


def _route_body(i1_ref, i2_ref, sel1_ref, sel2_ref, cnt1_ref, cnt2_ref,
                r1_out, r2_out, te_out):
    cnt1 = cnt1_ref[...]
    cnt = cnt1 + cnt2_ref[...]
    padded = jnp.ceil(cnt / P_TILE) * P_TILE
    cols = []
    acc = jnp.zeros((1, 1), _F32)
    for e in range(NUM_E):
        cols.append(acc)
        acc = acc + padded[0:1, e:e + 1]
    pad_off = jnp.concatenate(cols, axis=1)
    lane = jax.lax.broadcasted_iota(_I32, (N_TOK, NUM_E), 1)
    i1 = i1_ref[...]
    i2 = i2_ref[...]
    off1 = jnp.sum(jnp.where(lane == i1, pad_off, 0.0), axis=1, keepdims=True)
    off2 = jnp.sum(jnp.where(lane == i2, pad_off + cnt1, 0.0), axis=1,
                   keepdims=True)
    r1_out[...] = (off1 + sel1_ref[...].astype(_F32)).astype(_I32)
    r2_out[...] = (off2 + sel2_ref[...].astype(_F32)).astype(_I32)
    tpos = (jax.lax.broadcasted_iota(_I32, (1, N_PTILES), 1)
            .astype(_F32) * P_TILE)
    te = jnp.zeros((1, N_PTILES), _F32)
    for e in range(NUM_E):
        te = te + jnp.where(pad_off[0:1, e:e + 1] <= tpos, 1.0, 0.0)
    te_out[...] = jnp.clip(te - 1.0, 0, NUM_E - 1).astype(_I32)


def _expert_body(te_ref, z_ref, r1_ref, r2_ref, p1_ref, p2_ref, w1_ref,
                 w2_ref, out_ref):
    del te_ref
    h = jax.nn.gelu(_bdot(z_ref[...].astype(_BF16), w1_ref[0]))
    y = _bdot(h.astype(_BF16), w2_ref[0])
    slot = (jax.lax.broadcasted_iota(_I32, (P_TILE, N_TOK), 0)
            + pl.program_id(0) * P_TILE)
    oh1 = (r1_ref[...] == slot).astype(_BF16)
    oh2 = (r2_ref[...] == slot).astype(_BF16)
    p_tile = (_bdot(oh1, p1_ref[...].astype(_BF16))
              + _bdot(oh2, p2_ref[...].astype(_BF16)))
    out_ref[...] = p_tile * y.astype(_BF16).astype(_F32)


def _scores_body(tdf_ref, wdrug_ref, wrole_ref, et_ref, form_ref, out_ref):
    tdf = tdf_ref[...].astype(_BF16)
    dp = _bdot(tdf, wdrug_ref[...])
    dpb = dp.astype(_BF16)
    rt = jnp.tanh(_bdot(tdf, wrole_ref[...]))
    rtb = rt.astype(_BF16)
    etb = et_ref[...].astype(_BF16)
    dn = (((1,), (1,)), ((), ()))
    out_ref[...] = (
        jax.lax.dot_general(etb, dpb, dn, preferred_element_type=_F32)
        + jax.lax.dot_general(form_ref[...], rtb, dn,
                              preferred_element_type=_F32))


B_W = B_TGT // 16


def _dispatch_body(z_hbm, r1_hbm, r2_hbm, df_hbm, tidx_hbm, zs_hbm, tdf_hbm,
                   idx1_v, idx2_v, rows_v, tidx_v, trow_v, sem1, sem2, sem3):
    wid = lax.axis_index("s") * 2 + lax.axis_index("c")
    base = wid * TOK_W
    pltpu.sync_copy(r1_hbm.at[pl.ds(base, TOK_W)], idx1_v)
    pltpu.sync_copy(r2_hbm.at[pl.ds(base, TOK_W)], idx2_v)
    pltpu.sync_copy(z_hbm.at[pl.ds(base, TOK_W)], rows_v)
    c1 = pltpu.async_copy(rows_v, zs_hbm.at[idx1_v], sem1)
    c2 = pltpu.async_copy(rows_v, zs_hbm.at[idx2_v], sem2)

    @pl.when(wid < 16)
    def _():
        tbase = wid * B_W
        pltpu.sync_copy(tidx_hbm.at[pl.ds(tbase, B_W)], tidx_v)
        cg = pltpu.async_copy(df_hbm.at[tidx_v], trow_v, sem3)
        cg.wait()
        pltpu.sync_copy(trow_v, tdf_hbm.at[pl.ds(tbase, B_W)])

    c1.wait()
    c2.wait()


@functools.lru_cache(maxsize=None)
def _dispatch_sc():
    mesh = plsc.VectorSubcoreMesh(core_axis_name="c", subcore_axis_name="s")
    return pl.kernel(
        _dispatch_body,
        out_type=[
            jax.ShapeDtypeStruct((P_ROWS, D_MODEL), jnp.float32),
            jax.ShapeDtypeStruct((B_TGT, DRUG_DIM), jnp.float32),
        ],
        mesh=mesh,
        scratch_types=[
            pltpu.VMEM((TOK_W,), _I32),
            pltpu.VMEM((TOK_W,), _I32),
            pltpu.VMEM((TOK_W, D_MODEL), _F32),
            pltpu.VMEM((B_W,), _I32),
            pltpu.VMEM((B_W, DRUG_DIM), _F32),
            pltpu.SemaphoreType.DMA,
            pltpu.SemaphoreType.DMA,
            pltpu.SemaphoreType.DMA,
        ],
    )


def _combine_body(y_hbm, r1_hbm, r2_hbm, et_hbm, i1_v, i2_v, a_v, b_v, o_v,
                  sema, semb):
    wid = lax.axis_index("s") * 2 + lax.axis_index("c")
    for sub in range(TOK_W // SUB_W):
        base = wid * TOK_W + sub * SUB_W
        pltpu.sync_copy(r1_hbm.at[pl.ds(base, SUB_W)], i1_v)
        pltpu.sync_copy(r2_hbm.at[pl.ds(base, SUB_W)], i2_v)
        ca = pltpu.async_copy(y_hbm.at[i1_v], a_v, sema)
        cb = pltpu.async_copy(y_hbm.at[i2_v], b_v, semb)
        ca.wait()
        cb.wait()
        for j in range(SUB_W):
            arow = a_v.at[j]
            brow = b_v.at[j]
            orow = o_v.at[j]

            def chunk(k, _, arow=arow, brow=brow, orow=orow):
                sl = pl.ds(k * 16, 16)
                orow[sl] = arow[sl] + brow[sl]
                return 0

            lax.fori_loop(0, D_MODEL // 16, chunk, 0)
        pltpu.sync_copy(o_v, et_hbm.at[pl.ds(base, SUB_W)])


@functools.lru_cache(maxsize=None)
def _combine_sc():
    mesh = plsc.VectorSubcoreMesh(core_axis_name="c", subcore_axis_name="s")
    return pl.kernel(
        _combine_body,
        out_type=jax.ShapeDtypeStruct((N_TOK, D_MODEL), jnp.float32),
        mesh=mesh,
        scratch_types=[
            pltpu.VMEM((SUB_W,), _I32),
            pltpu.VMEM((SUB_W,), _I32),
            pltpu.VMEM((SUB_W, D_MODEL), _F32),
            pltpu.VMEM((SUB_W, D_MODEL), _F32),
            pltpu.VMEM((SUB_W, D_MODEL), _F32),
            pltpu.SemaphoreType.DMA,
            pltpu.SemaphoreType.DMA,
        ],
    )


def kernel(pillar_x, cross_floor_jaccard, drug_features, target_drug_indices,
           W_in, b_in, W_out, b_out, gamma, beta, W_form, W_role, W_stats,
           W_gate, W_fg, j_bias, W1, W2, W_drug):
    f32 = _F32
    b_in2 = b_in.reshape(1, D_MODEL)
    b_out2 = b_out.reshape(1, D_MODEL)
    gamma2 = gamma.reshape(1, D_MODEL)
    beta2 = beta.reshape(1, D_MODEL)
    jbias2 = j_bias.reshape(1, NUM_E)
    jac2 = cross_floor_jaccard.reshape(1, 1).astype(f32)

    full = lambda shape: pl.BlockSpec(shape, lambda i: (0,) * len(shape))
    tile = lambda w: pl.BlockSpec((TOK_TILE, w), lambda i: (i, 0))
    (zrb, form, gate_probs, i1, i2, p1b, p2b, sel1, sel2, cnt1, cnt2) = (
        pl.pallas_call(
            _smoother_body,
            grid=(N_TILES,),
            in_specs=[
                tile(D_MODEL),
                full((D_MODEL, D_MODEL)),
                full((1, D_MODEL)),
                full((D_MODEL, D_MODEL)),
                full((1, D_MODEL)),
                full((1, D_MODEL)),
                full((1, D_MODEL)),
                full((D_MODEL, D_FOOT)),
                full((D_MODEL, NUM_E)),
                full((D_FOOT, NUM_E)),
                full((2, NUM_E)),
                full((1, NUM_E)),
                full((1, 1)),
            ],
            out_specs=[
                tile(D_MODEL),
                tile(D_FOOT),
                tile(NUM_E),
                tile(1),
                tile(1),
                tile(1),
                tile(1),
                tile(1),
                tile(1),
                pl.BlockSpec((1, NUM_E), lambda i: (0, 0)),
                pl.BlockSpec((1, NUM_E), lambda i: (0, 0)),
            ],
            out_shape=[
                jax.ShapeDtypeStruct((N_TOK, D_MODEL), _BF16),
                jax.ShapeDtypeStruct((N_TOK, D_FOOT), _BF16),
                jax.ShapeDtypeStruct((N_TOK, NUM_E), f32),
                jax.ShapeDtypeStruct((N_TOK, 1), _I32),
                jax.ShapeDtypeStruct((N_TOK, 1), _I32),
                jax.ShapeDtypeStruct((N_TOK, 1), f32),
                jax.ShapeDtypeStruct((N_TOK, 1), f32),
                jax.ShapeDtypeStruct((N_TOK, 1), _I32),
                jax.ShapeDtypeStruct((N_TOK, 1), _I32),
                jax.ShapeDtypeStruct((1, NUM_E), f32),
                jax.ShapeDtypeStruct((1, NUM_E), f32),
            ],
            scratch_shapes=[pltpu.VMEM((2, NUM_E), f32)],
        )(pillar_x, W_in.astype(_BF16), b_in2, W_out.astype(_BF16), b_out2,
          gamma2, beta2, W_form.astype(_BF16), W_gate.astype(_BF16),
          W_fg.astype(_BF16), W_stats, jbias2, jac2))
    zr32 = zrb.astype(f32)

    r1, r2, te = pl.pallas_call(
        _route_body,
        grid=(1,),
        in_specs=[
            pl.BlockSpec((N_TOK, 1), lambda i: (0, 0)),
            pl.BlockSpec((N_TOK, 1), lambda i: (0, 0)),
            pl.BlockSpec((N_TOK, 1), lambda i: (0, 0)),
            pl.BlockSpec((N_TOK, 1), lambda i: (0, 0)),
            pl.BlockSpec((1, NUM_E), lambda i: (0, 0)),
            pl.BlockSpec((1, NUM_E), lambda i: (0, 0)),
        ],
        out_specs=[
            pl.BlockSpec((N_TOK, 1), lambda i: (0, 0)),
            pl.BlockSpec((N_TOK, 1), lambda i: (0, 0)),
            pl.BlockSpec((1, N_PTILES), lambda i: (0, 0)),
        ],
        out_shape=[
            jax.ShapeDtypeStruct((N_TOK, 1), _I32),
            jax.ShapeDtypeStruct((N_TOK, 1), _I32),
            jax.ShapeDtypeStruct((1, N_PTILES), _I32),
        ],
    )(i1, i2, sel1, sel2, cnt1, cnt2)

    r1f = r1.reshape(N_TOK)
    r2f = r2.reshape(N_TOK)
    dfb32 = drug_features.astype(_BF16).astype(f32)
    z_sorted, tdf = _dispatch_sc()(zr32, r1f, r2f, dfb32,
                                   target_drug_indices)

    w1b = W1.astype(_BF16)
    w2b = W2.astype(_BF16)
    y_sorted = pl.pallas_call(
        _expert_body,
        grid_spec=pltpu.PrefetchScalarGridSpec(
            num_scalar_prefetch=1,
            grid=(N_PTILES,),
            in_specs=[
                pl.BlockSpec((P_TILE, D_MODEL), lambda i, te: (i, 0)),
                pl.BlockSpec((1, N_TOK), lambda i, te: (0, 0)),
                pl.BlockSpec((1, N_TOK), lambda i, te: (0, 0)),
                pl.BlockSpec((N_TOK, 1), lambda i, te: (0, 0)),
                pl.BlockSpec((N_TOK, 1), lambda i, te: (0, 0)),
                pl.BlockSpec((1, D_MODEL, D_FF), lambda i, te: (te[i], 0, 0)),
                pl.BlockSpec((1, D_FF, D_MODEL), lambda i, te: (te[i], 0, 0)),
            ],
            out_specs=pl.BlockSpec((P_TILE, D_MODEL), lambda i, te: (i, 0)),
        ),
        out_shape=jax.ShapeDtypeStruct((P_ROWS, D_MODEL), f32),
    )(te.reshape(N_PTILES), z_sorted, r1.reshape(1, N_TOK),
      r2.reshape(1, N_TOK), p1b, p2b, w1b, w2b)

    expert_tensor = _combine_sc()(y_sorted, r1f, r2f)

    scores = pl.pallas_call(
        _scores_body,
        grid=(N_TILES,),
        in_specs=[
            full((B_TGT, DRUG_DIM)),
            full((DRUG_DIM, D_MODEL)),
            full((DRUG_DIM, D_FOOT)),
            tile(D_MODEL),
            tile(D_FOOT),
        ],
        out_specs=pl.BlockSpec((TOK_TILE, B_TGT), lambda i: (i, 0)),
        out_shape=jax.ShapeDtypeStruct((N_TOK, B_TGT), f32),
    )(tdf, W_drug.astype(_BF16), W_role.astype(_BF16), expert_tensor, form)

    return (scores, gate_probs, expert_tensor)
